# baseline (speedup 1.0000x reference)
import functools

import jax
import jax.numpy as jnp
from jax import lax
from jax.experimental import pallas as pl
from jax.experimental.pallas import tpu as pltpu
from jax.experimental.pallas import tpu_sc as plsc

D_MODEL = 1024
MAXLEN = 4096
NUM_ROWS = 4 * 4096
NC, NS, LANES = 2, 16, 16
NW = NC * NS
B_PER_W = NUM_ROWS // NW
CHUNK = 16
NCHUNK = B_PER_W // CHUNK
NBUF = 7
WINDOW = 4


def _sc_gather(idx, table):
    mesh = plsc.VectorSubcoreMesh(core_axis_name="c", subcore_axis_name="s")

    @functools.partial(
        pl.kernel,
        mesh=mesh,
        out_type=jax.ShapeDtypeStruct((NUM_ROWS, 8, 128), jnp.float32),
        scratch_types=[
            pltpu.VMEM((NCHUNK, CHUNK), jnp.int32),
            pltpu.VMEM((NBUF, CHUNK, 8, 128), jnp.float32),
            pltpu.SemaphoreType.DMA((NBUF,)),
            pltpu.SemaphoreType.DMA((NBUF,)),
        ],
    )
    def body(idx_hbm, table_hbm, out_hbm, idx_v, rows_v, g_sem, w_sem):
        wid = lax.axis_index("s") * NC + lax.axis_index("c")
        base = wid * B_PER_W

        pltpu.sync_copy(idx_hbm.at[wid], idx_v)

        for c in range(NCHUNK):
            for j in range(CHUNK // LANES):
                v = idx_v[c, pl.ds(j * LANES, LANES)]
                v = jnp.minimum(jnp.maximum(v, -MAXLEN), MAXLEN - 1) + MAXLEN
                idx_v[c, pl.ds(j * LANES, LANES)] = v

        def fire_gather(c):
            b = c % NBUF
            return pltpu.async_copy(
                table_hbm.at[idx_v.at[c]], rows_v.at[b], g_sem.at[b])

        gathers = {}
        writes = {}
        unwaited_writes = []
        for c in range(min(WINDOW, NCHUNK)):
            gathers[c] = fire_gather(c)
        for c in range(NCHUNK):
            b = c % NBUF
            gathers[c].wait()
            writes[c] = pltpu.async_copy(
                rows_v.at[b],
                out_hbm.at[pl.ds(base + c * CHUNK, CHUNK)],
                w_sem.at[b])
            unwaited_writes.append(c)
            n = c + WINDOW
            if n < NCHUNK:
                prev = n - NBUF
                if prev >= 0:
                    writes[prev].wait()
                    unwaited_writes.remove(prev)
                gathers[n] = fire_gather(n)
        for c in unwaited_writes:
            writes[c].wait()

    return body(idx, table)


ROWS_PER_STEP = 2048


TABLE_SLABS = 8
SLAB_ROWS = 2 * MAXLEN // TABLE_SLABS


def _tc_gather(idx, table2):
    grid = TABLE_SLABS + NUM_ROWS // ROWS_PER_STEP

    def body(s_ref, t2_ref, o_ref, t3_ref):
        i = pl.program_id(0)

        @pl.when(i < TABLE_SLABS)
        def _stage():
            base = i * SLAB_ROWS
            for k in range(SLAB_ROWS // 8):
                blk = t2_ref[pl.ds(8 * k, 8), :]
                t3_ref[pl.ds(base + 8 * k, 8)] = blk.reshape(8, 8, 128)

        @pl.when(i >= TABLE_SLABS)
        def _gather():
            r0 = (i - TABLE_SLABS) * ROWS_PER_STEP
            for g in range(ROWS_PER_STEP // 8):
                rows = []
                for j in range(8):
                    p = s_ref[r0 + g * 8 + j]
                    p = jnp.minimum(jnp.maximum(p, -MAXLEN),
                                    MAXLEN - 1) + MAXLEN
                    rows.append(t3_ref[p])
                blk = jnp.stack(rows, axis=0).reshape(8, D_MODEL)
                o_ref[pl.ds(g * 8, 8), :] = blk

    grid_spec = pltpu.PrefetchScalarGridSpec(
        num_scalar_prefetch=1,
        grid=(grid,),
        in_specs=[
            pl.BlockSpec((SLAB_ROWS, D_MODEL),
                         lambda i, s: (jnp.minimum(i, TABLE_SLABS - 1), 0)),
        ],
        out_specs=pl.BlockSpec(
            (ROWS_PER_STEP, D_MODEL),
            lambda i, s: (jnp.maximum(i - TABLE_SLABS, 0), 0)),
        scratch_shapes=[pltpu.VMEM((2 * MAXLEN, 8, 128), jnp.float32)],
    )
    return pl.pallas_call(
        body,
        grid_spec=grid_spec,
        out_shape=jax.ShapeDtypeStruct((NUM_ROWS, D_MODEL), jnp.float32),
        compiler_params=pltpu.CompilerParams(
            vmem_limit_bytes=100 * 1024 * 1024,
        ),
    )(idx, table2)


def kernel(pos_seq, pe_k):
    out = _tc_gather(pos_seq.reshape(NUM_ROWS), pe_k)
    return out.reshape(pos_seq.shape[0], pos_seq.shape[1], D_MODEL)


# speedup vs baseline: 1.0367x; 1.0367x over previous
import functools

import jax
import jax.numpy as jnp
from jax import lax
from jax.experimental import pallas as pl
from jax.experimental.pallas import tpu as pltpu
from jax.experimental.pallas import tpu_sc as plsc

D_MODEL = 1024
MAXLEN = 4096
NUM_ROWS = 4 * 4096
NC, NS, LANES = 2, 16, 16
NW = NC * NS
B_PER_W = NUM_ROWS // NW
CHUNK = 16
NCHUNK = B_PER_W // CHUNK
NBUF = 7
WINDOW = 4


def _sc_gather(idx, table):
    mesh = plsc.VectorSubcoreMesh(core_axis_name="c", subcore_axis_name="s")

    @functools.partial(
        pl.kernel,
        mesh=mesh,
        out_type=jax.ShapeDtypeStruct((NUM_ROWS, 8, 128), jnp.float32),
        scratch_types=[
            pltpu.VMEM((NCHUNK, CHUNK), jnp.int32),
            pltpu.VMEM((NBUF, CHUNK, 8, 128), jnp.float32),
            pltpu.SemaphoreType.DMA((NBUF,)),
            pltpu.SemaphoreType.DMA((NBUF,)),
        ],
    )
    def body(idx_hbm, table_hbm, out_hbm, idx_v, rows_v, g_sem, w_sem):
        wid = lax.axis_index("s") * NC + lax.axis_index("c")
        base = wid * B_PER_W

        pltpu.sync_copy(idx_hbm.at[wid], idx_v)

        for c in range(NCHUNK):
            for j in range(CHUNK // LANES):
                v = idx_v[c, pl.ds(j * LANES, LANES)]
                v = jnp.minimum(jnp.maximum(v, -MAXLEN), MAXLEN - 1) + MAXLEN
                idx_v[c, pl.ds(j * LANES, LANES)] = v

        def fire_gather(c):
            b = c % NBUF
            return pltpu.async_copy(
                table_hbm.at[idx_v.at[c]], rows_v.at[b], g_sem.at[b])

        gathers = {}
        writes = {}
        unwaited_writes = []
        for c in range(min(WINDOW, NCHUNK)):
            gathers[c] = fire_gather(c)
        for c in range(NCHUNK):
            b = c % NBUF
            gathers[c].wait()
            writes[c] = pltpu.async_copy(
                rows_v.at[b],
                out_hbm.at[pl.ds(base + c * CHUNK, CHUNK)],
                w_sem.at[b])
            unwaited_writes.append(c)
            n = c + WINDOW
            if n < NCHUNK:
                prev = n - NBUF
                if prev >= 0:
                    writes[prev].wait()
                    unwaited_writes.remove(prev)
                gathers[n] = fire_gather(n)
        for c in unwaited_writes:
            writes[c].wait()

    return body(idx, table)


ROWS_PER_STEP = 1024


TABLE_SLABS = 4
SLAB_ROWS = 2 * MAXLEN // TABLE_SLABS


def _tc_gather(idx, table2):
    grid = TABLE_SLABS + NUM_ROWS // ROWS_PER_STEP

    def body(s_ref, t2_ref, o_ref, t3_ref):
        i = pl.program_id(0)

        @pl.when(i < TABLE_SLABS)
        def _stage():
            base = i * SLAB_ROWS
            for k in range(SLAB_ROWS // 8):
                blk = t2_ref[pl.ds(8 * k, 8), :]
                t3_ref[pl.ds(base + 8 * k, 8)] = blk.reshape(8, 8, 128)

        @pl.when(i >= TABLE_SLABS)
        def _gather():
            r0 = (i - TABLE_SLABS) * ROWS_PER_STEP
            for g in range(ROWS_PER_STEP // 8):
                rows = []
                for j in range(8):
                    p = s_ref[r0 + g * 8 + j]
                    p = jnp.minimum(jnp.maximum(p, -MAXLEN),
                                    MAXLEN - 1) + MAXLEN
                    rows.append(t3_ref[p])
                blk = jnp.stack(rows, axis=0).reshape(8, D_MODEL)
                o_ref[pl.ds(g * 8, 8), :] = blk

    grid_spec = pltpu.PrefetchScalarGridSpec(
        num_scalar_prefetch=1,
        grid=(grid,),
        in_specs=[
            pl.BlockSpec((SLAB_ROWS, D_MODEL),
                         lambda i, s: (jnp.minimum(i, TABLE_SLABS - 1), 0)),
        ],
        out_specs=pl.BlockSpec(
            (ROWS_PER_STEP, D_MODEL),
            lambda i, s: (jnp.maximum(i - TABLE_SLABS, 0), 0)),
        scratch_shapes=[pltpu.VMEM((2 * MAXLEN, 8, 128), jnp.float32)],
    )
    return pl.pallas_call(
        body,
        grid_spec=grid_spec,
        out_shape=jax.ShapeDtypeStruct((NUM_ROWS, D_MODEL), jnp.float32),
        compiler_params=pltpu.CompilerParams(
            vmem_limit_bytes=100 * 1024 * 1024,
        ),
    )(idx, table2)


def kernel(pos_seq, pe_k):
    out = _tc_gather(pos_seq.reshape(NUM_ROWS), pe_k)
    return out.reshape(pos_seq.shape[0], pos_seq.shape[1], D_MODEL)
